# Initial kernel scaffold; baseline (speedup 1.0000x reference)
import jax
import jax.numpy as jnp
from jax.experimental import pallas as pl


def kernel(x, edge_index, kernel_id, W_v, g_v, b_v, W_q, g_q, b_q, codebook, W_out, g_o, b_o):
    raise NotImplementedError("write your pallas kernel here")


# speedup vs baseline: 2.4813x; 2.4813x over previous
import functools

import jax
import jax.numpy as jnp
from jax import lax
from jax.experimental import pallas as pl
from jax.experimental.pallas import tpu as pltpu
from jax.experimental.pallas import tpu_sc as plsc

N = 10000
E = 320000
K = 27
PLANES = 128
C = 256
CH = 128
NPAD = 10016
YCOLS = 32
EPAD = 327680
EPT_AB = EPAD // 32
NCH_AB = EPT_AB // 128
EPT_C = EPAD // 16
NCH_C = EPT_C // 128

_f32 = jnp.float32
_i32 = jnp.int32

_mesh = plsc.VectorSubcoreMesh(core_axis_name="c", subcore_axis_name="s")


def _tc1_body(x_ref, wv_ref, gv_ref, bv_ref, wq_ref, cb_ref,
              vlo_ref, vhi_ref, y_ref, cbs_ref):
    x = x_ref[...]
    z = jnp.dot(x, wv_ref[...], preferred_element_type=_f32)
    mu = jnp.mean(z, 0, keepdims=True)
    zc = z - mu
    var = jnp.mean(zc * zc, 0, keepdims=True)
    v = jnp.maximum(gv_ref[...] * zc * lax.rsqrt(var + 1e-5) + bv_ref[...], 0.0)
    zpad = jnp.zeros((NPAD - N, CH), _f32)
    vlo_ref[...] = jnp.concatenate([v[:, :CH], zpad], 0)
    vhi_ref[...] = jnp.concatenate([v[:, CH:], zpad], 0)
    y = lax.dot_general(x, wq_ref[...], (((1,), (1,)), ((), ())),
                        preferred_element_type=_f32)
    ypad = jnp.zeros((N, YCOLS - K), _f32)
    ytail = jnp.zeros((NPAD - N, YCOLS), _f32)
    y_ref[...] = jnp.concatenate(
        [jnp.concatenate([y, ypad], 1), ytail], 0)
    cbs = jnp.sum(cb_ref[...], -1)
    cbs_ref[...] = jnp.concatenate(
        [cbs, jnp.zeros((2, YCOLS - K), _f32)], 1)


def _tc1(x, W_v, g_v, b_v, W_q, codebook):
    return pl.pallas_call(
        _tc1_body,
        out_shape=(
            jax.ShapeDtypeStruct((NPAD, CH), _f32),
            jax.ShapeDtypeStruct((NPAD, CH), _f32),
            jax.ShapeDtypeStruct((NPAD, YCOLS), _f32),
            jax.ShapeDtypeStruct((2, YCOLS), _f32),
        ),
    )(x, W_v, g_v, b_v, W_q, codebook)


def _tc2_body(qp_ref, gq_ref, bq_ref, q_ref):
    q = qp_ref[0, :] + qp_ref[1, :]
    mu = jnp.mean(q)
    qc = q - mu
    var = jnp.mean(qc * qc)
    qn = jnp.maximum(gq_ref[0] * qc * lax.rsqrt(var + 1e-5) + bq_ref[0], 0.0)
    q_ref[...] = jnp.concatenate([qn, jnp.zeros((NPAD - N,), _f32)])


def _tc2(qpart, g_q, b_q):
    return pl.pallas_call(
        _tc2_body,
        out_shape=jax.ShapeDtypeStruct((NPAD,), _f32),
    )(qpart, g_q, b_q)


def _tc3_body(cp_ref, w0_ref, w1_ref):
    cp = cp_ref[...]
    c0 = cp[0, 0] + cp[1, 0]
    c1 = cp[0, 1] + cp[1, 1]
    mx = jnp.maximum(c0, c1)
    e0 = jnp.exp(c0 - mx)
    e1 = jnp.exp(c1 - mx)
    inv = 1.0 / (e0 + e1)
    w0_ref[...] = e0 * inv
    w1_ref[...] = e1 * inv


def _tc3(cpart):
    return pl.pallas_call(
        _tc3_body,
        out_shape=(jax.ShapeDtypeStruct((N,), _f32),
                   jax.ShapeDtypeStruct((N,), _f32)),
    )(cpart)


def _tc4_body(alo_ref, ahi_ref, wo_ref, go_ref, bo_ref, x_ref, out_ref):
    z = (jnp.dot(alo_ref[...], wo_ref[0:CH, :], preferred_element_type=_f32)
         + jnp.dot(ahi_ref[...], wo_ref[CH:C, :], preferred_element_type=_f32))
    mu = jnp.mean(z, 0, keepdims=True)
    zc = z - mu
    var = jnp.mean(zc * zc, 0, keepdims=True)
    o = jnp.maximum(go_ref[...] * zc * lax.rsqrt(var + 1e-5) + bo_ref[...], 0.0)
    out_ref[...] = jnp.maximum(o + x_ref[...], 0.0)


def _tc4(alo, ahi, W_out, g_o, b_o, x):
    return pl.pallas_call(
        _tc4_body,
        out_shape=jax.ShapeDtypeStruct((N, PLANES), _f32),
    )(alo, ahi, W_out, g_o, b_o, x)


@functools.partial(
    pl.kernel, mesh=_mesh,
    out_type=jax.ShapeDtypeStruct((2, N), _f32),
    scratch_types=[
        pltpu.VMEM((EPT_AB,), _i32),
        pltpu.VMEM((EPT_AB,), _i32),
        pltpu.VMEM((NCH_AB, 128), _i32),
        pltpu.VMEM((EPT_AB,), _i32),
        pltpu.VMEM((NCH_AB, 128), _f32),
        pltpu.VMEM_SHARED((N,), _f32),
        pltpu.SemaphoreType.DMA,
    ],
)
def _sc_qraw(yflat, srcp, kidp, dst2d, zeros1, out,
             src_v, kid_v, dst_v, flat_v, vals_v, acc, sem):
    c = lax.axis_index("c")
    s = lax.axis_index("s")
    wid = s * 2 + c
    base = wid * EPT_AB

    @pl.when(s == 0)
    def _():
        pltpu.sync_copy(zeros1, acc)

    plsc.subcore_barrier()

    pltpu.sync_copy(srcp.at[pl.ds(base, EPT_AB)], src_v)
    pltpu.sync_copy(kidp.at[pl.ds(base, EPT_AB)], kid_v)
    pltpu.sync_copy(dst2d.at[pl.ds(wid * NCH_AB, NCH_AB)], dst_v)

    def flat_body(i, carry):
        sl = pl.ds(i * 16, 16)
        flat_v[sl] = src_v[sl] * YCOLS + kid_v[sl]
        return carry

    lax.fori_loop(0, EPT_AB // 16, flat_body, 0)

    def chunk_body(j, carry):
        idx = flat_v.at[pl.ds(j * 128, 128)]
        pltpu.async_copy(yflat.at[idx], vals_v.at[j], sem).wait()
        pltpu.sync_copy(vals_v.at[j], acc.at[dst_v.at[j]], add=True)
        return carry

    lax.fori_loop(0, NCH_AB, chunk_body, 0)

    plsc.subcore_barrier()

    @pl.when(s == 0)
    def _():
        @pl.when(c == 0)
        def _():
            pltpu.sync_copy(acc, out.at[0])

        @pl.when(c == 1)
        def _():
            pltpu.sync_copy(acc, out.at[1])


@functools.partial(
    pl.kernel, mesh=_mesh,
    out_type=jax.ShapeDtypeStruct((2, 2, N), _f32),
    scratch_types=[
        pltpu.VMEM((EPT_AB,), _i32),
        pltpu.VMEM((EPT_AB,), _i32),
        pltpu.VMEM((NCH_AB, 128), _i32),
        pltpu.VMEM((NCH_AB, 128), _f32),
        pltpu.VMEM((128,), _f32),
        pltpu.VMEM((128,), _f32),
        pltpu.VMEM((128,), _f32),
        pltpu.VMEM((128,), _f32),
        pltpu.VMEM_SHARED((N,), _f32),
        pltpu.VMEM_SHARED((N,), _f32),
        pltpu.SemaphoreType.DMA,
    ],
)
def _sc_choice(qext, srcp, kidp, dst2d, cbs0, cbs1, zeros1, out,
               src_v, kid_v, dst_v, qg_v, p0_v, p1_v, cb0g_v, cb1g_v,
               acc0, acc1, sem):
    c = lax.axis_index("c")
    s = lax.axis_index("s")
    wid = s * 2 + c
    base = wid * EPT_AB

    @pl.when(s == 0)
    def _():
        pltpu.sync_copy(zeros1, acc0)
        pltpu.sync_copy(zeros1, acc1)

    plsc.subcore_barrier()

    pltpu.sync_copy(srcp.at[pl.ds(base, EPT_AB)], src_v)
    pltpu.sync_copy(kidp.at[pl.ds(base, EPT_AB)], kid_v)
    pltpu.sync_copy(dst2d.at[pl.ds(wid * NCH_AB, NCH_AB)], dst_v)

    def chunk_body(j, carry):
        idx = src_v.at[pl.ds(j * 128, 128)]
        kididx = kid_v.at[pl.ds(j * 128, 128)]
        pltpu.async_copy(qext.at[idx], qg_v.at[j], sem).wait()
        pltpu.async_copy(cbs0.at[kididx], cb0g_v, sem).wait()
        pltpu.async_copy(cbs1.at[kididx], cb1g_v, sem).wait()
        for g in range(8):
            sl = pl.ds(g * 16, 16)
            qg = qg_v[j, sl]
            p0_v[sl] = qg * cb0g_v[sl]
            p1_v[sl] = qg * cb1g_v[sl]
        pltpu.sync_copy(p0_v, acc0.at[dst_v.at[j]], add=True)
        pltpu.sync_copy(p1_v, acc1.at[dst_v.at[j]], add=True)
        return carry

    lax.fori_loop(0, NCH_AB, chunk_body, 0)

    plsc.subcore_barrier()

    @pl.when(s == 0)
    def _():
        @pl.when(c == 0)
        def _():
            pltpu.sync_copy(acc0, out.at[0, 0])
            pltpu.sync_copy(acc1, out.at[0, 1])

        @pl.when(c == 1)
        def _():
            pltpu.sync_copy(acc0, out.at[1, 0])
            pltpu.sync_copy(acc1, out.at[1, 1])


@functools.partial(
    pl.kernel, mesh=_mesh,
    out_type=(jax.ShapeDtypeStruct((N, CH), _f32),
              jax.ShapeDtypeStruct((N, CH), _f32)),
    scratch_types=[
        pltpu.VMEM((3, 128), _i32),
        pltpu.VMEM((144,), _f32),
        pltpu.VMEM((144,), _f32),
        pltpu.VMEM((128, CH), _f32),
        pltpu.VMEM((K, CH), _f32),
        pltpu.VMEM((K, CH), _f32),
        pltpu.VMEM_SHARED((N, CH), _f32),
        pltpu.SemaphoreType.DMA,
    ],
)
def _sc_main(vlo, vhi, edata, w0t, w1t, cb_lo, cb_hi, zeros2, out_lo, out_hi,
             ed_v, w0g, w1g, vrow, cb0_v, cb1_v, acc, sem):
    c = lax.axis_index("c")
    s = lax.axis_index("s")

    @pl.when(s == 0)
    def _():
        pltpu.sync_copy(zeros2, acc)

    @pl.when(c == 0)
    def _():
        pltpu.sync_copy(cb_lo.at[0], cb0_v)
        pltpu.sync_copy(cb_lo.at[1], cb1_v)

    @pl.when(c == 1)
    def _():
        pltpu.sync_copy(cb_hi.at[0], cb0_v)
        pltpu.sync_copy(cb_hi.at[1], cb1_v)

    plsc.subcore_barrier()

    def chunk_body(j, carry):
        pltpu.sync_copy(edata.at[s * NCH_C + j], ed_v)
        srcidx = ed_v.at[0]
        dstidx = ed_v.at[2]

        @pl.when(c == 0)
        def _():
            pltpu.async_copy(vlo.at[srcidx], vrow, sem).wait()

        @pl.when(c == 1)
        def _():
            pltpu.async_copy(vhi.at[srcidx], vrow, sem).wait()

        pltpu.async_copy(w0t.at[dstidx], w0g.at[pl.ds(0, 128)], sem).wait()
        pltpu.async_copy(w1t.at[dstidx], w1g.at[pl.ds(0, 128)], sem).wait()

        def edge_body(e, carry2):
            k = ed_v[1, pl.ds(e, 16)][0]
            w0s = jnp.full((16,), w0g[pl.ds(e, 16)][0], _f32)
            w1s = jnp.full((16,), w1g[pl.ds(e, 16)][0], _f32)
            for g in range(8):
                sl = pl.ds(g * 16, 16)
                omega = w0s * cb0_v[k, sl] + w1s * cb1_v[k, sl]
                vrow[e, sl] = vrow[e, sl] * omega
            return carry2

        lax.fori_loop(0, 128, edge_body, 0)

        pltpu.sync_copy(vrow, acc.at[dstidx], add=True)
        return carry

    lax.fori_loop(0, NCH_C, chunk_body, 0)

    plsc.subcore_barrier()

    @pl.when(s == 0)
    def _():
        @pl.when(c == 0)
        def _():
            pltpu.sync_copy(acc, out_lo)

        @pl.when(c == 1)
        def _():
            pltpu.sync_copy(acc, out_hi)


def kernel(x, edge_index, kernel_id, W_v, g_v, b_v, W_q, g_q, b_q,
           codebook, W_out, g_o, b_o):
    src = edge_index[0]
    dst = edge_index[1]
    pad = EPAD - E
    srcp = jnp.concatenate([src, jnp.full((pad,), N, _i32)])
    kidp = jnp.concatenate([kernel_id, jnp.zeros((pad,), _i32)])
    dstp = jnp.concatenate([dst, jnp.zeros((pad,), _i32)])
    dst2d = dstp.reshape(EPAD // 128, 128)
    zeros1 = jnp.zeros((N,), _f32)
    zeros2 = jnp.zeros((N, CH), _f32)

    vlo, vhi, ypad, cbs = _tc1(x, W_v, g_v, b_v, W_q, codebook)
    yflat = ypad.reshape(-1)

    qpart = _sc_qraw(yflat, srcp, kidp, dst2d, zeros1)
    qext = _tc2(qpart, g_q, b_q)
    cpart = _sc_choice(qext, srcp, kidp, dst2d, cbs[0], cbs[1], zeros1)
    w0, w1 = _tc3(cpart)

    cb_lo = codebook[:, :, :CH]
    cb_hi = codebook[:, :, CH:]
    edata = jnp.stack([srcp.reshape(-1, 128), kidp.reshape(-1, 128),
                       dstp.reshape(-1, 128)], axis=1)
    alo, ahi = _sc_main(vlo, vhi, edata, w0, w1,
                        cb_lo, cb_hi, zeros2)
    return _tc4(alo, ahi, W_out, g_o, b_o, x)


# speedup vs baseline: 5.5829x; 2.2500x over previous
import functools

import jax
import jax.numpy as jnp
from jax import lax
from jax.experimental import pallas as pl
from jax.experimental.pallas import tpu as pltpu
from jax.experimental.pallas import tpu_sc as plsc

N = 10000
E = 320000
K = 27
PLANES = 128
C = 256
CH = 128
NPAD = 10016
YCOLS = 32
EPAD = 327680
EPT_AB = EPAD // 32
NCH_AB = EPT_AB // 128
EPT_C = EPAD // 16
NCH_C = EPT_C // 128
SEG = 16

_f32 = jnp.float32
_i32 = jnp.int32

_mesh = plsc.VectorSubcoreMesh(core_axis_name="c", subcore_axis_name="s")


def _tc1_body(x_ref, wv_ref, gv_ref, bv_ref, wq_ref, cb_ref,
              vlo_ref, vhi_ref, y_ref, cbs_ref):
    x = x_ref[...]
    z = jnp.dot(x, wv_ref[...], preferred_element_type=_f32)
    mu = jnp.mean(z, 0, keepdims=True)
    zc = z - mu
    var = jnp.mean(zc * zc, 0, keepdims=True)
    v = jnp.maximum(gv_ref[...] * zc * lax.rsqrt(var + 1e-5) + bv_ref[...], 0.0)
    zpad = jnp.zeros((NPAD - N, CH), _f32)
    vlo_ref[...] = jnp.concatenate([v[:, :CH], zpad], 0)
    vhi_ref[...] = jnp.concatenate([v[:, CH:], zpad], 0)
    y = lax.dot_general(x, wq_ref[...], (((1,), (1,)), ((), ())),
                        preferred_element_type=_f32)
    ypad = jnp.zeros((N, YCOLS - K), _f32)
    ytail = jnp.zeros((NPAD - N, YCOLS), _f32)
    y_ref[...] = jnp.concatenate(
        [jnp.concatenate([y, ypad], 1), ytail], 0)
    cbs = jnp.sum(cb_ref[...], -1)
    cbs_ref[...] = jnp.concatenate(
        [cbs, jnp.zeros((2, YCOLS - K), _f32)], 1)


def _tc1(x, W_v, g_v, b_v, W_q, codebook):
    return pl.pallas_call(
        _tc1_body,
        out_shape=(
            jax.ShapeDtypeStruct((NPAD, CH), _f32),
            jax.ShapeDtypeStruct((NPAD, CH), _f32),
            jax.ShapeDtypeStruct((NPAD, YCOLS), _f32),
            jax.ShapeDtypeStruct((2, YCOLS), _f32),
        ),
    )(x, W_v, g_v, b_v, W_q, codebook)


def _tc2_body(qp_ref, gq_ref, bq_ref, q_ref):
    q = qp_ref[0, :] + qp_ref[1, :]
    mu = jnp.mean(q)
    qc = q - mu
    var = jnp.mean(qc * qc)
    qn = jnp.maximum(gq_ref[0] * qc * lax.rsqrt(var + 1e-5) + bq_ref[0], 0.0)
    q_ref[...] = jnp.concatenate([qn, jnp.zeros((NPAD - N,), _f32)])


def _tc2(qpart, g_q, b_q):
    return pl.pallas_call(
        _tc2_body,
        out_shape=jax.ShapeDtypeStruct((NPAD,), _f32),
    )(qpart, g_q, b_q)


def _tc3_body(cp_ref, cbs_ref, w0_ref, w1_ref):
    svals = cp_ref[0] + cp_ref[1]
    cb = cbs_ref[...]
    cm = jnp.dot(cb[:, 0:K], svals, preferred_element_type=_f32)
    c0 = cm[0]
    c1 = cm[1]
    mx = jnp.maximum(c0, c1)
    e0 = jnp.exp(c0 - mx)
    e1 = jnp.exp(c1 - mx)
    inv = 1.0 / (e0 + e1)
    w0_ref[...] = e0 * inv
    w1_ref[...] = e1 * inv


def _tc3(cpart, cbs):
    return pl.pallas_call(
        _tc3_body,
        out_shape=(jax.ShapeDtypeStruct((N,), _f32),
                   jax.ShapeDtypeStruct((N,), _f32)),
    )(cpart, cbs)


def _tc4_body(alo_ref, ahi_ref, wo_ref, go_ref, bo_ref, x_ref, out_ref):
    z = (jnp.dot(alo_ref[...], wo_ref[0:CH, :], preferred_element_type=_f32)
         + jnp.dot(ahi_ref[...], wo_ref[CH:C, :], preferred_element_type=_f32))
    mu = jnp.mean(z, 0, keepdims=True)
    zc = z - mu
    var = jnp.mean(zc * zc, 0, keepdims=True)
    o = jnp.maximum(go_ref[...] * zc * lax.rsqrt(var + 1e-5) + bo_ref[...], 0.0)
    out_ref[...] = jnp.maximum(o + x_ref[...], 0.0)


def _tc4(alo, ahi, W_out, g_o, b_o, x):
    return pl.pallas_call(
        _tc4_body,
        out_shape=jax.ShapeDtypeStruct((N, PLANES), _f32),
    )(alo, ahi, W_out, g_o, b_o, x)


@functools.partial(
    pl.kernel, mesh=_mesh,
    out_type=jax.ShapeDtypeStruct((2, N), _f32),
    scratch_types=[
        pltpu.VMEM((EPT_AB,), _i32),
        pltpu.VMEM((EPT_AB,), _i32),
        pltpu.VMEM((NCH_AB, 128), _i32),
        pltpu.VMEM((EPT_AB,), _i32),
        pltpu.VMEM((NCH_AB, 128), _f32),
        pltpu.VMEM_SHARED((N,), _f32),
        pltpu.SemaphoreType.DMA,
    ],
)
def _sc_qraw(yflat, srcp, kidp, dst2d, zeros1, out,
             src_v, kid_v, dst_v, flat_v, vals_v, acc, sem):
    c = lax.axis_index("c")
    s = lax.axis_index("s")
    wid = s * 2 + c
    base = wid * EPT_AB

    @pl.when(s == 0)
    def _():
        pltpu.sync_copy(zeros1, acc)

    plsc.subcore_barrier()

    pltpu.sync_copy(srcp.at[pl.ds(base, EPT_AB)], src_v)
    pltpu.sync_copy(kidp.at[pl.ds(base, EPT_AB)], kid_v)
    pltpu.sync_copy(dst2d.at[pl.ds(wid * NCH_AB, NCH_AB)], dst_v)

    def flat_body(i, carry):
        sl = pl.ds(i * 16, 16)
        flat_v[sl] = src_v[sl] * YCOLS + kid_v[sl]
        return carry

    lax.fori_loop(0, EPT_AB // 16, flat_body, 0)

    def chunk_body(j, carry):
        idx = flat_v.at[pl.ds(j * 128, 128)]
        pltpu.async_copy(yflat.at[idx], vals_v.at[j], sem).wait()
        pltpu.sync_copy(vals_v.at[j], acc.at[dst_v.at[j]], add=True)
        return carry

    lax.fori_loop(0, NCH_AB, chunk_body, 0)

    plsc.subcore_barrier()

    @pl.when(s == 0)
    def _():
        @pl.when(c == 0)
        def _():
            pltpu.sync_copy(acc, out.at[0])

        @pl.when(c == 1)
        def _():
            pltpu.sync_copy(acc, out.at[1])


@functools.partial(
    pl.kernel, mesh=_mesh,
    out_type=jax.ShapeDtypeStruct((2, K * N), _f32),
    scratch_types=[
        pltpu.VMEM((EPT_AB,), _i32),
        pltpu.VMEM((EPT_AB,), _i32),
        pltpu.VMEM((EPT_AB,), _i32),
        pltpu.VMEM((2, 128), _f32),
        pltpu.VMEM((2, 128), _i32),
        pltpu.VMEM_SHARED((K * N,), _f32),
        pltpu.SemaphoreType.DMA,
        pltpu.SemaphoreType.DMA,
    ],
)
def _sc_choice(qext, srcp, kidp, dstp, zeros27, out,
               src_v, kid_v, dst_v, qg, fidx, acc27, sg0, sg1):
    c = lax.axis_index("c")
    s = lax.axis_index("s")
    wid = s * 2 + c
    base = wid * EPT_AB

    @pl.when(s == 0)
    def _():
        pltpu.sync_copy(zeros27, acc27)

    pltpu.sync_copy(srcp.at[pl.ds(base, EPT_AB)], src_v)
    pltpu.sync_copy(kidp.at[pl.ds(base, EPT_AB)], kid_v)
    pltpu.sync_copy(dstp.at[pl.ds(base, EPT_AB)], dst_v)

    plsc.subcore_barrier()

    def fire(j, b):
        pltpu.async_copy(qext.at[src_v.at[pl.ds(j * 128, 128)]],
                         qg.at[b], sg0 if b == 0 else sg1)

    def wait_g(b):
        pltpu.make_async_copy(qext.at[pl.ds(0, 128)], qg.at[b],
                              sg0 if b == 0 else sg1).wait()

    def mkidx(j, b):
        for g in range(8):
            sl = pl.ds(g * 16, 16)
            esl = pl.ds(j * 128 + g * 16, 16)
            fidx[b, sl] = kid_v[esl] * N + dst_v[esl]

    fire(0, 0)

    def pair_body(t, carry):
        j0 = 2 * t
        j1 = 2 * t + 1
        fire(j1, 1)
        mkidx(j0, 0)
        wait_g(0)
        pltpu.sync_copy(qg.at[0], acc27.at[fidx.at[0]], add=True)

        @pl.when(t < NCH_AB // 2 - 1)
        def _():
            fire(j0 + 2, 0)

        mkidx(j1, 1)
        wait_g(1)
        pltpu.sync_copy(qg.at[1], acc27.at[fidx.at[1]], add=True)
        return carry

    lax.fori_loop(0, NCH_AB // 2, pair_body, 0)

    plsc.subcore_barrier()

    @pl.when(s == 0)
    def _():
        @pl.when(c == 0)
        def _():
            pltpu.sync_copy(acc27, out.at[0])

        @pl.when(c == 1)
        def _():
            pltpu.sync_copy(acc27, out.at[1])


@functools.partial(
    pl.kernel, mesh=_mesh,
    out_type=(jax.ShapeDtypeStruct((N, CH), _f32),
              jax.ShapeDtypeStruct((N, CH), _f32)),
    scratch_types=[
        pltpu.VMEM((SEG * 2 + 1, 128), _i32),
        pltpu.VMEM((SEG, 128), _i32),
        pltpu.VMEM((144,), _f32),
        pltpu.VMEM((144,), _f32),
        pltpu.VMEM((144,), _f32),
        pltpu.VMEM((144,), _f32),
        pltpu.VMEM((128, CH), _f32),
        pltpu.VMEM((128, CH), _f32),
        pltpu.VMEM((K, CH), _f32),
        pltpu.VMEM((K, CH), _f32),
        pltpu.VMEM_SHARED((N, CH), _f32),
        pltpu.SemaphoreType.DMA,
        pltpu.SemaphoreType.DMA,
        pltpu.SemaphoreType.DMA,
        pltpu.SemaphoreType.DMA,
    ],
)
def _sc_main(vlo, vhi, sk2d, dst2d, w0t, w1t, cb_lo, cb_hi, zeros2,
             out_lo, out_hi,
             sk_v, dst_v, w0g0, w1g0, w0g1, w1g1, vrow0, vrow1,
             cb0_v, cb1_v, acc, sg0, sg1, ss0, ss1):
    c = lax.axis_index("c")
    s = lax.axis_index("s")

    @pl.when(s == 0)
    def _():
        pltpu.sync_copy(zeros2, acc)

    @pl.when(c == 0)
    def _():
        pltpu.sync_copy(cb_lo.at[0], cb0_v)
        pltpu.sync_copy(cb_lo.at[1], cb1_v)

    @pl.when(c == 1)
    def _():
        pltpu.sync_copy(cb_hi.at[0], cb0_v)
        pltpu.sync_copy(cb_hi.at[1], cb1_v)

    plsc.subcore_barrier()

    def fire_data(t, vrow_b, w0g_b, w1g_b, sg):
        srcidx = sk_v.at[2 * t]

        @pl.when(c == 0)
        def _():
            pltpu.async_copy(vlo.at[srcidx], vrow_b, sg)

        @pl.when(c == 1)
        def _():
            pltpu.async_copy(vhi.at[srcidx], vrow_b, sg)

        didx = dst_v.at[t]
        pltpu.async_copy(w0t.at[didx], w0g_b.at[pl.ds(0, 128)], sg)
        pltpu.async_copy(w1t.at[didx], w1g_b.at[pl.ds(0, 128)], sg)

    def wait_data(vrow_b, w0g_b, w1g_b, sg):
        pltpu.make_async_copy(vlo.at[pl.ds(0, 128)], vrow_b, sg).wait()
        pltpu.make_async_copy(w0t.at[pl.ds(0, 128)],
                              w0g_b.at[pl.ds(0, 128)], sg).wait()
        pltpu.make_async_copy(w0t.at[pl.ds(0, 128)],
                              w1g_b.at[pl.ds(0, 128)], sg).wait()

    def compute(t, vrow_b, w0g_b, w1g_b):
        def edge_body(e, carry2):
            k = sk_v[2 * t + 1, pl.ds(e, 16)][0]
            w0s = jnp.full((16,), w0g_b[pl.ds(e, 16)][0], _f32)
            w1s = jnp.full((16,), w1g_b[pl.ds(e, 16)][0], _f32)
            for g in range(8):
                sl = pl.ds(g * 16, 16)
                omega = w0s * cb0_v[k, sl] + w1s * cb1_v[k, sl]
                vrow_b[e, sl] = vrow_b[e, sl] * omega
            return carry2

        lax.fori_loop(0, 128, edge_body, 0)

    def fire_scat(t, vrow_b, ss):
        pltpu.async_copy(vrow_b, acc.at[dst_v.at[t]], ss, add=True)

    def wait_scat(vrow_b, ss):
        pltpu.make_async_copy(vrow_b, acc.at[pl.ds(0, 128)], ss).wait()

    def seg_body(gseg, carry):
        segrow = s * NCH_C + gseg * SEG
        pltpu.sync_copy(sk2d.at[pl.ds(2 * segrow, SEG * 2)],
                        sk_v.at[pl.ds(0, SEG * 2)])
        pltpu.sync_copy(dst2d.at[pl.ds(segrow, SEG)], dst_v)
        fire_data(0, vrow0, w0g0, w1g0, sg0)

        def pair_body(t2, carry2):
            t0 = 2 * t2
            t1 = 2 * t2 + 1
            fire_data(t1, vrow1, w0g1, w1g1, sg1)
            wait_data(vrow0, w0g0, w1g0, sg0)
            compute(t0, vrow0, w0g0, w1g0)
            fire_scat(t0, vrow0, ss0)

            @pl.when(t2 < SEG // 2 - 1)
            def _():
                wait_scat(vrow0, ss0)
                fire_data(t0 + 2, vrow0, w0g0, w1g0, sg0)

            wait_data(vrow1, w0g1, w1g1, sg1)
            compute(t1, vrow1, w0g1, w1g1)
            fire_scat(t1, vrow1, ss1)

            @pl.when(t2 < SEG // 2 - 1)
            def _():
                wait_scat(vrow1, ss1)

            return carry2

        lax.fori_loop(0, SEG // 2, pair_body, 0)
        wait_scat(vrow0, ss0)
        wait_scat(vrow1, ss1)
        return carry

    lax.fori_loop(0, NCH_C // SEG, seg_body, 0)

    plsc.subcore_barrier()

    @pl.when(s == 0)
    def _():
        @pl.when(c == 0)
        def _():
            pltpu.sync_copy(acc, out_lo)

        @pl.when(c == 1)
        def _():
            pltpu.sync_copy(acc, out_hi)


def kernel(x, edge_index, kernel_id, W_v, g_v, b_v, W_q, g_q, b_q,
           codebook, W_out, g_o, b_o):
    src = edge_index[0]
    dst = edge_index[1]
    pad = EPAD - E
    srcp = jnp.concatenate([src, jnp.full((pad,), N, _i32)])
    kidp = jnp.concatenate([kernel_id, jnp.zeros((pad,), _i32)])
    dstp = jnp.concatenate([dst, jnp.zeros((pad,), _i32)])
    dst2d = dstp.reshape(EPAD // 128, 128)
    zeros1 = jnp.zeros((N,), _f32)
    zeros2 = jnp.zeros((N, CH), _f32)

    vlo, vhi, ypad, cbs = _tc1(x, W_v, g_v, b_v, W_q, codebook)
    yflat = ypad.reshape(-1)

    zeros27 = jnp.zeros((K * N,), _f32)
    qpart = _sc_qraw(yflat, srcp, kidp, dst2d, zeros1)
    qext = _tc2(qpart, g_q, b_q)
    cpart = _sc_choice(qext, srcp, kidp, dstp, zeros27)
    w0, w1 = _tc3(cpart.reshape(2, K, N), cbs)

    cb_lo = codebook[:, :, :CH]
    cb_hi = codebook[:, :, CH:]
    sk2d = jnp.stack([srcp.reshape(-1, 128), kidp.reshape(-1, 128)],
                     axis=1).reshape(-1, 128)
    alo, ahi = _sc_main(vlo, vhi, sk2d, dst2d, w0, w1,
                        cb_lo, cb_hi, zeros2)
    return _tc4(alo, ahi, W_out, g_o, b_o, x)


# speedup vs baseline: 6.3433x; 1.1362x over previous
import functools

import jax
import jax.numpy as jnp
from jax import lax
from jax.experimental import pallas as pl
from jax.experimental.pallas import tpu as pltpu
from jax.experimental.pallas import tpu_sc as plsc

N = 10000
E = 320000
K = 27
PLANES = 128
C = 256
CH = 128
NPAD = 10016
YCOLS = 32
EPAD = 327680
EPT_AB = EPAD // 32
NCH_AB = EPT_AB // 128
EPT_C = EPAD // 16
NCH_C = EPT_C // 128
SEG = 16

_f32 = jnp.float32
_i32 = jnp.int32

_mesh = plsc.VectorSubcoreMesh(core_axis_name="c", subcore_axis_name="s")


def _tc1_body(x_ref, wv_ref, gv_ref, bv_ref, wq_ref, cb_ref,
              vlo_ref, vhi_ref, y_ref, cbs_ref):
    x = x_ref[...]
    z = jnp.dot(x, wv_ref[...], preferred_element_type=_f32)
    mu = jnp.mean(z, 0, keepdims=True)
    zc = z - mu
    var = jnp.mean(zc * zc, 0, keepdims=True)
    v = jnp.maximum(gv_ref[...] * zc * lax.rsqrt(var + 1e-5) + bv_ref[...], 0.0)
    zpad = jnp.zeros((NPAD - N, CH), _f32)
    vlo_ref[...] = jnp.concatenate([v[:, :CH], zpad], 0)
    vhi_ref[...] = jnp.concatenate([v[:, CH:], zpad], 0)
    y = lax.dot_general(x, wq_ref[...], (((1,), (1,)), ((), ())),
                        preferred_element_type=_f32)
    ypad = jnp.zeros((N, YCOLS - K), _f32)
    ytail = jnp.zeros((NPAD - N, YCOLS), _f32)
    y_ref[...] = jnp.concatenate(
        [jnp.concatenate([y, ypad], 1), ytail], 0)
    cbs = jnp.sum(cb_ref[...], -1)
    cbs_ref[...] = jnp.concatenate(
        [cbs, jnp.zeros((2, YCOLS - K), _f32)], 1)


def _tc1(x, W_v, g_v, b_v, W_q, codebook):
    return pl.pallas_call(
        _tc1_body,
        out_shape=(
            jax.ShapeDtypeStruct((NPAD, CH), _f32),
            jax.ShapeDtypeStruct((NPAD, CH), _f32),
            jax.ShapeDtypeStruct((NPAD, YCOLS), _f32),
            jax.ShapeDtypeStruct((2, YCOLS), _f32),
        ),
    )(x, W_v, g_v, b_v, W_q, codebook)


def _tc2_body(qp_ref, gq_ref, bq_ref, q_ref):
    q = qp_ref[0, :] + qp_ref[1, :]
    mu = jnp.mean(q)
    qc = q - mu
    var = jnp.mean(qc * qc)
    qn = jnp.maximum(gq_ref[0] * qc * lax.rsqrt(var + 1e-5) + bq_ref[0], 0.0)
    q_ref[...] = jnp.concatenate([qn, jnp.zeros((NPAD - N,), _f32)])


def _tc2(qpart, g_q, b_q):
    return pl.pallas_call(
        _tc2_body,
        out_shape=jax.ShapeDtypeStruct((NPAD,), _f32),
    )(qpart, g_q, b_q)


def _tc3_body(cp_ref, cbs_ref, w0_ref, w1_ref):
    svals = cp_ref[0] + cp_ref[1]
    cb = cbs_ref[...]
    cm = jnp.dot(cb[:, 0:K], svals, preferred_element_type=_f32)
    c0 = cm[0]
    c1 = cm[1]
    mx = jnp.maximum(c0, c1)
    e0 = jnp.exp(c0 - mx)
    e1 = jnp.exp(c1 - mx)
    inv = 1.0 / (e0 + e1)
    w0_ref[...] = e0 * inv
    w1_ref[...] = e1 * inv


def _tc3(cpart, cbs):
    return pl.pallas_call(
        _tc3_body,
        out_shape=(jax.ShapeDtypeStruct((N,), _f32),
                   jax.ShapeDtypeStruct((N,), _f32)),
    )(cpart, cbs)


def _tc4_body(alo_ref, ahi_ref, wo_ref, go_ref, bo_ref, x_ref, out_ref):
    z = (jnp.dot(alo_ref[...], wo_ref[0:CH, :], preferred_element_type=_f32)
         + jnp.dot(ahi_ref[...], wo_ref[CH:C, :], preferred_element_type=_f32))
    mu = jnp.mean(z, 0, keepdims=True)
    zc = z - mu
    var = jnp.mean(zc * zc, 0, keepdims=True)
    o = jnp.maximum(go_ref[...] * zc * lax.rsqrt(var + 1e-5) + bo_ref[...], 0.0)
    out_ref[...] = jnp.maximum(o + x_ref[...], 0.0)


def _tc4(alo, ahi, W_out, g_o, b_o, x):
    return pl.pallas_call(
        _tc4_body,
        out_shape=jax.ShapeDtypeStruct((N, PLANES), _f32),
    )(alo, ahi, W_out, g_o, b_o, x)


@functools.partial(
    pl.kernel, mesh=_mesh,
    out_type=jax.ShapeDtypeStruct((2, N), _f32),
    scratch_types=[
        pltpu.VMEM((EPT_AB,), _i32),
        pltpu.VMEM((EPT_AB,), _i32),
        pltpu.VMEM((NCH_AB, 128), _i32),
        pltpu.VMEM((EPT_AB,), _i32),
        pltpu.VMEM((NCH_AB, 128), _f32),
        pltpu.VMEM_SHARED((N,), _f32),
        pltpu.SemaphoreType.DMA,
    ],
)
def _sc_qraw(yflat, srcp, kidp, dst2d, zeros1, out,
             src_v, kid_v, dst_v, flat_v, vals_v, acc, sem):
    c = lax.axis_index("c")
    s = lax.axis_index("s")
    wid = s * 2 + c
    base = wid * EPT_AB

    @pl.when(s == 0)
    def _():
        pltpu.sync_copy(zeros1, acc)

    plsc.subcore_barrier()

    pltpu.sync_copy(srcp.at[pl.ds(base, EPT_AB)], src_v)
    pltpu.sync_copy(kidp.at[pl.ds(base, EPT_AB)], kid_v)
    pltpu.sync_copy(dst2d.at[pl.ds(wid * NCH_AB, NCH_AB)], dst_v)

    def flat_body(i, carry):
        sl = pl.ds(i * 16, 16)
        flat_v[sl] = src_v[sl] * YCOLS + kid_v[sl]
        return carry

    lax.fori_loop(0, EPT_AB // 16, flat_body, 0)

    def chunk_body(j, carry):
        idx = flat_v.at[pl.ds(j * 128, 128)]
        pltpu.async_copy(yflat.at[idx], vals_v.at[j], sem).wait()
        pltpu.sync_copy(vals_v.at[j], acc.at[dst_v.at[j]], add=True)
        return carry

    lax.fori_loop(0, NCH_AB, chunk_body, 0)

    plsc.subcore_barrier()

    @pl.when(s == 0)
    def _():
        @pl.when(c == 0)
        def _():
            pltpu.sync_copy(acc, out.at[0])

        @pl.when(c == 1)
        def _():
            pltpu.sync_copy(acc, out.at[1])


@functools.partial(
    pl.kernel, mesh=_mesh,
    out_type=jax.ShapeDtypeStruct((2, K * N), _f32),
    scratch_types=[
        pltpu.VMEM((EPT_AB,), _i32),
        pltpu.VMEM((EPT_AB,), _i32),
        pltpu.VMEM((EPT_AB,), _i32),
        pltpu.VMEM((2, 128), _f32),
        pltpu.VMEM((2, 128), _i32),
        pltpu.VMEM_SHARED((K * N,), _f32),
        pltpu.SemaphoreType.DMA,
        pltpu.SemaphoreType.DMA,
    ],
)
def _sc_choice(qext, srcp, kidp, dstp, zeros27, out,
               src_v, kid_v, dst_v, qg, fidx, acc27, sg0, sg1):
    c = lax.axis_index("c")
    s = lax.axis_index("s")
    wid = s * 2 + c
    base = wid * EPT_AB

    @pl.when(s == 0)
    def _():
        pltpu.sync_copy(zeros27, acc27)

    pltpu.sync_copy(srcp.at[pl.ds(base, EPT_AB)], src_v)
    pltpu.sync_copy(kidp.at[pl.ds(base, EPT_AB)], kid_v)
    pltpu.sync_copy(dstp.at[pl.ds(base, EPT_AB)], dst_v)

    plsc.subcore_barrier()

    def fire(j, b):
        pltpu.async_copy(qext.at[src_v.at[pl.ds(j * 128, 128)]],
                         qg.at[b], sg0 if b == 0 else sg1)

    def wait_g(b):
        pltpu.make_async_copy(qext.at[pl.ds(0, 128)], qg.at[b],
                              sg0 if b == 0 else sg1).wait()

    def mkidx(j, b):
        for g in range(8):
            sl = pl.ds(g * 16, 16)
            esl = pl.ds(j * 128 + g * 16, 16)
            fidx[b, sl] = kid_v[esl] * N + dst_v[esl]

    fire(0, 0)

    def pair_body(t, carry):
        j0 = 2 * t
        j1 = 2 * t + 1
        fire(j1, 1)
        mkidx(j0, 0)
        wait_g(0)
        pltpu.sync_copy(qg.at[0], acc27.at[fidx.at[0]], add=True)

        @pl.when(t < NCH_AB // 2 - 1)
        def _():
            fire(j0 + 2, 0)

        mkidx(j1, 1)
        wait_g(1)
        pltpu.sync_copy(qg.at[1], acc27.at[fidx.at[1]], add=True)
        return carry

    lax.fori_loop(0, NCH_AB // 2, pair_body, 0)

    plsc.subcore_barrier()

    @pl.when(s == 0)
    def _():
        @pl.when(c == 0)
        def _():
            pltpu.sync_copy(acc27, out.at[0])

        @pl.when(c == 1)
        def _():
            pltpu.sync_copy(acc27, out.at[1])


@functools.partial(
    pl.kernel, mesh=_mesh,
    out_type=(jax.ShapeDtypeStruct((N, CH), _f32),
              jax.ShapeDtypeStruct((N, CH), _f32)),
    scratch_types=[
        pltpu.VMEM((SEG * 2 + 1, 128), _i32),
        pltpu.VMEM((SEG, 128), _i32),
        pltpu.VMEM((144,), _f32),
        pltpu.VMEM((144,), _f32),
        pltpu.VMEM((144,), _f32),
        pltpu.VMEM((144,), _f32),
        pltpu.VMEM((128, CH), _f32),
        pltpu.VMEM((128, CH), _f32),
        pltpu.VMEM((K, CH), _f32),
        pltpu.VMEM((K, CH), _f32),
        pltpu.VMEM_SHARED((N, CH), _f32),
        pltpu.SemaphoreType.DMA,
        pltpu.SemaphoreType.DMA,
        pltpu.SemaphoreType.DMA,
        pltpu.SemaphoreType.DMA,
    ],
)
def _sc_main(vlo, vhi, sk2d, dst2d, w0t, w1t, cb_lo, cb_hi, zeros2,
             out_lo, out_hi,
             sk_v, dst_v, w0g0, w1g0, w0g1, w1g1, vrow0, vrow1,
             cb0_v, cb1_v, acc, sg0, sg1, ss0, ss1):
    c = lax.axis_index("c")
    s = lax.axis_index("s")

    @pl.when(s == 0)
    def _():
        pltpu.sync_copy(zeros2, acc)

    @pl.when(c == 0)
    def _():
        pltpu.sync_copy(cb_lo.at[0], cb0_v)
        pltpu.sync_copy(cb_lo.at[1], cb1_v)

    @pl.when(c == 1)
    def _():
        pltpu.sync_copy(cb_hi.at[0], cb0_v)
        pltpu.sync_copy(cb_hi.at[1], cb1_v)

    plsc.subcore_barrier()

    def fire_data(t, vrow_b, w0g_b, w1g_b, sg):
        srcidx = sk_v.at[2 * t]

        @pl.when(c == 0)
        def _():
            pltpu.async_copy(vlo.at[srcidx], vrow_b, sg)

        @pl.when(c == 1)
        def _():
            pltpu.async_copy(vhi.at[srcidx], vrow_b, sg)

        didx = dst_v.at[t]
        pltpu.async_copy(w0t.at[didx], w0g_b.at[pl.ds(0, 128)], sg)
        pltpu.async_copy(w1t.at[didx], w1g_b.at[pl.ds(0, 128)], sg)

    def wait_data(vrow_b, w0g_b, w1g_b, sg):
        pltpu.make_async_copy(vlo.at[pl.ds(0, 128)], vrow_b, sg).wait()
        pltpu.make_async_copy(w0t.at[pl.ds(0, 128)],
                              w0g_b.at[pl.ds(0, 128)], sg).wait()
        pltpu.make_async_copy(w0t.at[pl.ds(0, 128)],
                              w1g_b.at[pl.ds(0, 128)], sg).wait()

    def compute(t, vrow_b, w0g_b, w1g_b):
        def grp_body(eg, carry2):
            kv = sk_v[2 * t + 1, pl.ds(eg * 16, 16)]
            w0v = w0g_b[pl.ds(eg * 16, 16)]
            w1v = w1g_b[pl.ds(eg * 16, 16)]
            for lane in range(16):
                e = eg * 16 + lane
                k = kv[lane]
                w0s = jnp.full((16,), w0v[lane], _f32)
                w1s = jnp.full((16,), w1v[lane], _f32)
                for g in range(8):
                    sl = pl.ds(g * 16, 16)
                    omega = w0s * cb0_v[k, sl] + w1s * cb1_v[k, sl]
                    vrow_b[e, sl] = vrow_b[e, sl] * omega
            return carry2

        lax.fori_loop(0, 8, grp_body, 0)

    def fire_scat(t, vrow_b, ss):
        pltpu.async_copy(vrow_b, acc.at[dst_v.at[t]], ss, add=True)

    def wait_scat(vrow_b, ss):
        pltpu.make_async_copy(vrow_b, acc.at[pl.ds(0, 128)], ss).wait()

    def seg_body(gseg, carry):
        segrow = s * NCH_C + gseg * SEG
        pltpu.sync_copy(sk2d.at[pl.ds(2 * segrow, SEG * 2)],
                        sk_v.at[pl.ds(0, SEG * 2)])
        pltpu.sync_copy(dst2d.at[pl.ds(segrow, SEG)], dst_v)
        fire_data(0, vrow0, w0g0, w1g0, sg0)

        def pair_body(t2, carry2):
            t0 = 2 * t2
            t1 = 2 * t2 + 1
            fire_data(t1, vrow1, w0g1, w1g1, sg1)
            wait_data(vrow0, w0g0, w1g0, sg0)
            compute(t0, vrow0, w0g0, w1g0)
            fire_scat(t0, vrow0, ss0)

            @pl.when(t2 < SEG // 2 - 1)
            def _():
                wait_scat(vrow0, ss0)
                fire_data(t0 + 2, vrow0, w0g0, w1g0, sg0)

            wait_data(vrow1, w0g1, w1g1, sg1)
            compute(t1, vrow1, w0g1, w1g1)
            fire_scat(t1, vrow1, ss1)

            @pl.when(t2 < SEG // 2 - 1)
            def _():
                wait_scat(vrow1, ss1)

            return carry2

        lax.fori_loop(0, SEG // 2, pair_body, 0)
        wait_scat(vrow0, ss0)
        wait_scat(vrow1, ss1)
        return carry

    lax.fori_loop(0, NCH_C // SEG, seg_body, 0)

    plsc.subcore_barrier()

    @pl.when(s == 0)
    def _():
        @pl.when(c == 0)
        def _():
            pltpu.sync_copy(acc, out_lo)

        @pl.when(c == 1)
        def _():
            pltpu.sync_copy(acc, out_hi)


def kernel(x, edge_index, kernel_id, W_v, g_v, b_v, W_q, g_q, b_q,
           codebook, W_out, g_o, b_o):
    src = edge_index[0]
    dst = edge_index[1]
    pad = EPAD - E
    srcp = jnp.concatenate([src, jnp.full((pad,), N, _i32)])
    kidp = jnp.concatenate([kernel_id, jnp.zeros((pad,), _i32)])
    dstp = jnp.concatenate([dst, jnp.zeros((pad,), _i32)])
    dst2d = dstp.reshape(EPAD // 128, 128)
    zeros1 = jnp.zeros((N,), _f32)
    zeros2 = jnp.zeros((N, CH), _f32)

    vlo, vhi, ypad, cbs = _tc1(x, W_v, g_v, b_v, W_q, codebook)
    yflat = ypad.reshape(-1)

    zeros27 = jnp.zeros((K * N,), _f32)
    qpart = _sc_qraw(yflat, srcp, kidp, dst2d, zeros1)
    qext = _tc2(qpart, g_q, b_q)
    cpart = _sc_choice(qext, srcp, kidp, dstp, zeros27)
    w0, w1 = _tc3(cpart.reshape(2, K, N), cbs)

    cb_lo = codebook[:, :, :CH]
    cb_hi = codebook[:, :, CH:]
    sk2d = jnp.stack([srcp.reshape(-1, 128), kidp.reshape(-1, 128)],
                     axis=1).reshape(-1, 128)
    alo, ahi = _sc_main(vlo, vhi, sk2d, dst2d, w0, w1,
                        cb_lo, cb_hi, zeros2)
    return _tc4(alo, ahi, W_out, g_o, b_o, x)


# speedup vs baseline: 6.4591x; 1.0183x over previous
import functools

import jax
import jax.numpy as jnp
from jax import lax
from jax.experimental import pallas as pl
from jax.experimental.pallas import tpu as pltpu
from jax.experimental.pallas import tpu_sc as plsc

N = 10000
E = 320000
K = 27
PLANES = 128
C = 256
CH = 128
NPAD = 10016
YCOLS = 32
EPAD = 327680
EPT_AB = EPAD // 32
NCH_AB = EPT_AB // 128
EPT_C = EPAD // 16
NCH_C = EPT_C // 128
SEG = 16

_f32 = jnp.float32
_i32 = jnp.int32

_mesh = plsc.VectorSubcoreMesh(core_axis_name="c", subcore_axis_name="s")


def _tc1_body(x_ref, wv_ref, gv_ref, bv_ref, wq_ref, cb_ref,
              vlo_ref, vhi_ref, y_ref, cbs_ref):
    x = x_ref[...]
    z = jnp.dot(x, wv_ref[...], preferred_element_type=_f32)
    mu = jnp.mean(z, 0, keepdims=True)
    zc = z - mu
    var = jnp.mean(zc * zc, 0, keepdims=True)
    v = jnp.maximum(gv_ref[...] * zc * lax.rsqrt(var + 1e-5) + bv_ref[...], 0.0)
    zpad = jnp.zeros((NPAD - N, CH), _f32)
    vlo_ref[...] = jnp.concatenate([v[:, :CH], zpad], 0)
    vhi_ref[...] = jnp.concatenate([v[:, CH:], zpad], 0)
    y = lax.dot_general(x, wq_ref[...], (((1,), (1,)), ((), ())),
                        preferred_element_type=_f32)
    ypad = jnp.zeros((N, YCOLS - K), _f32)
    ytail = jnp.zeros((NPAD - N, YCOLS), _f32)
    y_ref[...] = jnp.concatenate(
        [jnp.concatenate([y, ypad], 1), ytail], 0)
    cbs = jnp.sum(cb_ref[...], -1)
    cbs_ref[...] = jnp.concatenate(
        [cbs, jnp.zeros((2, YCOLS - K), _f32)], 1)


def _tc1(x, W_v, g_v, b_v, W_q, codebook):
    return pl.pallas_call(
        _tc1_body,
        out_shape=(
            jax.ShapeDtypeStruct((NPAD, CH), _f32),
            jax.ShapeDtypeStruct((NPAD, CH), _f32),
            jax.ShapeDtypeStruct((NPAD, YCOLS), _f32),
            jax.ShapeDtypeStruct((2, YCOLS), _f32),
        ),
    )(x, W_v, g_v, b_v, W_q, codebook)


def _tc2_body(qp_ref, gq_ref, bq_ref, q_ref):
    q = qp_ref[0, :] + qp_ref[1, :]
    mu = jnp.mean(q)
    qc = q - mu
    var = jnp.mean(qc * qc)
    qn = jnp.maximum(gq_ref[0] * qc * lax.rsqrt(var + 1e-5) + bq_ref[0], 0.0)
    q_ref[...] = jnp.concatenate([qn, jnp.zeros((NPAD - N,), _f32)])


def _tc2(qpart, g_q, b_q):
    return pl.pallas_call(
        _tc2_body,
        out_shape=jax.ShapeDtypeStruct((NPAD,), _f32),
    )(qpart, g_q, b_q)


def _tc3_body(cp_ref, cbs_ref, w0_ref, w1_ref):
    svals = cp_ref[0] + cp_ref[1]
    cb = cbs_ref[...]
    cm = jnp.dot(cb[:, 0:K], svals, preferred_element_type=_f32)
    c0 = cm[0]
    c1 = cm[1]
    mx = jnp.maximum(c0, c1)
    e0 = jnp.exp(c0 - mx)
    e1 = jnp.exp(c1 - mx)
    inv = 1.0 / (e0 + e1)
    w0_ref[...] = e0 * inv
    w1_ref[...] = e1 * inv


def _tc3(cpart, cbs):
    return pl.pallas_call(
        _tc3_body,
        out_shape=(jax.ShapeDtypeStruct((N,), _f32),
                   jax.ShapeDtypeStruct((N,), _f32)),
    )(cpart, cbs)


def _tc4_body(alo_ref, ahi_ref, wo_ref, go_ref, bo_ref, x_ref, out_ref):
    z = (jnp.dot(alo_ref[...], wo_ref[0:CH, :], preferred_element_type=_f32)
         + jnp.dot(ahi_ref[...], wo_ref[CH:C, :], preferred_element_type=_f32))
    mu = jnp.mean(z, 0, keepdims=True)
    zc = z - mu
    var = jnp.mean(zc * zc, 0, keepdims=True)
    o = jnp.maximum(go_ref[...] * zc * lax.rsqrt(var + 1e-5) + bo_ref[...], 0.0)
    out_ref[...] = jnp.maximum(o + x_ref[...], 0.0)


def _tc4(alo, ahi, W_out, g_o, b_o, x):
    return pl.pallas_call(
        _tc4_body,
        out_shape=jax.ShapeDtypeStruct((N, PLANES), _f32),
    )(alo, ahi, W_out, g_o, b_o, x)


@functools.partial(
    pl.kernel, mesh=_mesh,
    out_type=jax.ShapeDtypeStruct((2, N), _f32),
    scratch_types=[
        pltpu.VMEM((EPT_AB,), _i32),
        pltpu.VMEM((EPT_AB,), _i32),
        pltpu.VMEM((NCH_AB, 128), _i32),
        pltpu.VMEM((EPT_AB,), _i32),
        pltpu.VMEM((2, 128), _f32),
        pltpu.VMEM_SHARED((N,), _f32),
        pltpu.SemaphoreType.DMA,
        pltpu.SemaphoreType.DMA,
    ],
)
def _sc_qraw(yflat, srcp, kidp, dst2d, zeros1, out,
             src_v, kid_v, dst_v, flat_v, vals_v, acc, sg0, sg1):
    c = lax.axis_index("c")
    s = lax.axis_index("s")
    wid = s * 2 + c
    base = wid * EPT_AB

    @pl.when(s == 0)
    def _():
        pltpu.sync_copy(zeros1, acc)

    pltpu.sync_copy(srcp.at[pl.ds(base, EPT_AB)], src_v)
    pltpu.sync_copy(kidp.at[pl.ds(base, EPT_AB)], kid_v)
    pltpu.sync_copy(dst2d.at[pl.ds(wid * NCH_AB, NCH_AB)], dst_v)

    def flat_body(i, carry):
        sl = pl.ds(i * 16, 16)
        flat_v[sl] = src_v[sl] * YCOLS + kid_v[sl]
        return carry

    lax.fori_loop(0, EPT_AB // 16, flat_body, 0)

    plsc.subcore_barrier()

    def fire(j, b):
        pltpu.async_copy(yflat.at[flat_v.at[pl.ds(j * 128, 128)]],
                         vals_v.at[b], sg0 if b == 0 else sg1)

    def wait_g(b):
        pltpu.make_async_copy(yflat.at[pl.ds(0, 128)], vals_v.at[b],
                              sg0 if b == 0 else sg1).wait()

    fire(0, 0)

    def pair_body(t, carry):
        j0 = 2 * t
        j1 = 2 * t + 1
        fire(j1, 1)
        wait_g(0)
        pltpu.sync_copy(vals_v.at[0], acc.at[dst_v.at[j0]], add=True)

        @pl.when(t < NCH_AB // 2 - 1)
        def _():
            fire(j0 + 2, 0)

        wait_g(1)
        pltpu.sync_copy(vals_v.at[1], acc.at[dst_v.at[j1]], add=True)
        return carry

    lax.fori_loop(0, NCH_AB // 2, pair_body, 0)

    plsc.subcore_barrier()

    @pl.when(s == 0)
    def _():
        @pl.when(c == 0)
        def _():
            pltpu.sync_copy(acc, out.at[0])

        @pl.when(c == 1)
        def _():
            pltpu.sync_copy(acc, out.at[1])


@functools.partial(
    pl.kernel, mesh=_mesh,
    out_type=jax.ShapeDtypeStruct((2, K * N), _f32),
    scratch_types=[
        pltpu.VMEM((EPT_AB,), _i32),
        pltpu.VMEM((EPT_AB,), _i32),
        pltpu.VMEM((EPT_AB,), _i32),
        pltpu.VMEM((2, 128), _f32),
        pltpu.VMEM((2, 128), _i32),
        pltpu.VMEM_SHARED((K * N,), _f32),
        pltpu.SemaphoreType.DMA,
        pltpu.SemaphoreType.DMA,
    ],
)
def _sc_choice(qext, srcp, kidp, dstp, zeros27, out,
               src_v, kid_v, dst_v, qg, fidx, acc27, sg0, sg1):
    c = lax.axis_index("c")
    s = lax.axis_index("s")
    wid = s * 2 + c
    base = wid * EPT_AB

    @pl.when(s == 0)
    def _():
        pltpu.sync_copy(zeros27, acc27)

    pltpu.sync_copy(srcp.at[pl.ds(base, EPT_AB)], src_v)
    pltpu.sync_copy(kidp.at[pl.ds(base, EPT_AB)], kid_v)
    pltpu.sync_copy(dstp.at[pl.ds(base, EPT_AB)], dst_v)

    plsc.subcore_barrier()

    def fire(j, b):
        pltpu.async_copy(qext.at[src_v.at[pl.ds(j * 128, 128)]],
                         qg.at[b], sg0 if b == 0 else sg1)

    def wait_g(b):
        pltpu.make_async_copy(qext.at[pl.ds(0, 128)], qg.at[b],
                              sg0 if b == 0 else sg1).wait()

    def mkidx(j, b):
        for g in range(8):
            sl = pl.ds(g * 16, 16)
            esl = pl.ds(j * 128 + g * 16, 16)
            fidx[b, sl] = kid_v[esl] * N + dst_v[esl]

    fire(0, 0)

    def pair_body(t, carry):
        j0 = 2 * t
        j1 = 2 * t + 1
        fire(j1, 1)
        mkidx(j0, 0)
        wait_g(0)
        pltpu.sync_copy(qg.at[0], acc27.at[fidx.at[0]], add=True)

        @pl.when(t < NCH_AB // 2 - 1)
        def _():
            fire(j0 + 2, 0)

        mkidx(j1, 1)
        wait_g(1)
        pltpu.sync_copy(qg.at[1], acc27.at[fidx.at[1]], add=True)
        return carry

    lax.fori_loop(0, NCH_AB // 2, pair_body, 0)

    plsc.subcore_barrier()

    @pl.when(s == 0)
    def _():
        @pl.when(c == 0)
        def _():
            pltpu.sync_copy(acc27, out.at[0])

        @pl.when(c == 1)
        def _():
            pltpu.sync_copy(acc27, out.at[1])


@functools.partial(
    pl.kernel, mesh=_mesh,
    out_type=(jax.ShapeDtypeStruct((N, CH), _f32),
              jax.ShapeDtypeStruct((N, CH), _f32)),
    scratch_types=[
        pltpu.VMEM((SEG * 2 + 1, 128), _i32),
        pltpu.VMEM((SEG, 128), _i32),
        pltpu.VMEM((144,), _f32),
        pltpu.VMEM((144,), _f32),
        pltpu.VMEM((144,), _f32),
        pltpu.VMEM((144,), _f32),
        pltpu.VMEM((128, CH), _f32),
        pltpu.VMEM((128, CH), _f32),
        pltpu.VMEM((K, CH), _f32),
        pltpu.VMEM((K, CH), _f32),
        pltpu.VMEM_SHARED((N, CH), _f32),
        pltpu.SemaphoreType.DMA,
        pltpu.SemaphoreType.DMA,
        pltpu.SemaphoreType.DMA,
        pltpu.SemaphoreType.DMA,
    ],
)
def _sc_main(vlo, vhi, sk2d, dst2d, w0t, w1t, cb_lo, cb_hi, zeros2,
             out_lo, out_hi,
             sk_v, dst_v, w0g0, w1g0, w0g1, w1g1, vrow0, vrow1,
             cb0_v, cb1_v, acc, sg0, sg1, ss0, ss1):
    c = lax.axis_index("c")
    s = lax.axis_index("s")

    @pl.when(s == 0)
    def _():
        pltpu.sync_copy(zeros2, acc)

    @pl.when(c == 0)
    def _():
        pltpu.sync_copy(cb_lo.at[0], cb0_v)
        pltpu.sync_copy(cb_lo.at[1], cb1_v)

    @pl.when(c == 1)
    def _():
        pltpu.sync_copy(cb_hi.at[0], cb0_v)
        pltpu.sync_copy(cb_hi.at[1], cb1_v)

    plsc.subcore_barrier()

    def fire_data(t, vrow_b, w0g_b, w1g_b, sg):
        srcidx = sk_v.at[2 * t]

        @pl.when(c == 0)
        def _():
            pltpu.async_copy(vlo.at[srcidx], vrow_b, sg)

        @pl.when(c == 1)
        def _():
            pltpu.async_copy(vhi.at[srcidx], vrow_b, sg)

        didx = dst_v.at[t]
        pltpu.async_copy(w0t.at[didx], w0g_b.at[pl.ds(0, 128)], sg)
        pltpu.async_copy(w1t.at[didx], w1g_b.at[pl.ds(0, 128)], sg)

    def wait_data(vrow_b, w0g_b, w1g_b, sg):
        pltpu.make_async_copy(vlo.at[pl.ds(0, 128)], vrow_b, sg).wait()
        pltpu.make_async_copy(w0t.at[pl.ds(0, 128)],
                              w0g_b.at[pl.ds(0, 128)], sg).wait()
        pltpu.make_async_copy(w0t.at[pl.ds(0, 128)],
                              w1g_b.at[pl.ds(0, 128)], sg).wait()

    def compute(t, vrow_b, w0g_b, w1g_b):
        def grp_body(eg, carry2):
            kv = sk_v[2 * t + 1, pl.ds(eg * 16, 16)]
            w0v = w0g_b[pl.ds(eg * 16, 16)]
            w1v = w1g_b[pl.ds(eg * 16, 16)]
            for lane in range(16):
                e = eg * 16 + lane
                k = kv[lane]
                w0s = jnp.full((16,), w0v[lane], _f32)
                w1s = jnp.full((16,), w1v[lane], _f32)
                for g in range(8):
                    sl = pl.ds(g * 16, 16)
                    omega = w0s * cb0_v[k, sl] + w1s * cb1_v[k, sl]
                    vrow_b[e, sl] = vrow_b[e, sl] * omega
            return carry2

        lax.fori_loop(0, 8, grp_body, 0)

    def fire_scat(t, vrow_b, ss):
        pltpu.async_copy(vrow_b, acc.at[dst_v.at[t]], ss, add=True)

    def wait_scat(vrow_b, ss):
        pltpu.make_async_copy(vrow_b, acc.at[pl.ds(0, 128)], ss).wait()

    def seg_body(gseg, carry):
        segrow = s * NCH_C + gseg * SEG
        pltpu.sync_copy(sk2d.at[pl.ds(2 * segrow, SEG * 2)],
                        sk_v.at[pl.ds(0, SEG * 2)])
        pltpu.sync_copy(dst2d.at[pl.ds(segrow, SEG)], dst_v)
        fire_data(0, vrow0, w0g0, w1g0, sg0)

        def pair_body(t2, carry2):
            t0 = 2 * t2
            t1 = 2 * t2 + 1
            fire_data(t1, vrow1, w0g1, w1g1, sg1)
            wait_data(vrow0, w0g0, w1g0, sg0)
            compute(t0, vrow0, w0g0, w1g0)
            fire_scat(t0, vrow0, ss0)

            @pl.when(t2 < SEG // 2 - 1)
            def _():
                wait_scat(vrow0, ss0)
                fire_data(t0 + 2, vrow0, w0g0, w1g0, sg0)

            wait_data(vrow1, w0g1, w1g1, sg1)
            compute(t1, vrow1, w0g1, w1g1)
            fire_scat(t1, vrow1, ss1)

            @pl.when(t2 < SEG // 2 - 1)
            def _():
                wait_scat(vrow1, ss1)

            return carry2

        lax.fori_loop(0, SEG // 2, pair_body, 0)
        wait_scat(vrow0, ss0)
        wait_scat(vrow1, ss1)
        return carry

    lax.fori_loop(0, NCH_C // SEG, seg_body, 0)

    plsc.subcore_barrier()

    @pl.when(s == 0)
    def _():
        @pl.when(c == 0)
        def _():
            pltpu.sync_copy(acc, out_lo)

        @pl.when(c == 1)
        def _():
            pltpu.sync_copy(acc, out_hi)


def kernel(x, edge_index, kernel_id, W_v, g_v, b_v, W_q, g_q, b_q,
           codebook, W_out, g_o, b_o):
    src = edge_index[0]
    dst = edge_index[1]
    pad = EPAD - E
    srcp = jnp.concatenate([src, jnp.full((pad,), N, _i32)])
    kidp = jnp.concatenate([kernel_id, jnp.zeros((pad,), _i32)])
    dstp = jnp.concatenate([dst, jnp.zeros((pad,), _i32)])
    dst2d = dstp.reshape(EPAD // 128, 128)
    zeros1 = jnp.zeros((N,), _f32)
    zeros2 = jnp.zeros((N, CH), _f32)

    vlo, vhi, ypad, cbs = _tc1(x, W_v, g_v, b_v, W_q, codebook)
    yflat = ypad.reshape(-1)

    zeros27 = jnp.zeros((K * N,), _f32)
    qpart = _sc_qraw(yflat, srcp, kidp, dst2d, zeros1)
    qext = _tc2(qpart, g_q, b_q)
    cpart = _sc_choice(qext, srcp, kidp, dstp, zeros27)
    w0, w1 = _tc3(cpart.reshape(2, K, N), cbs)

    cb_lo = codebook[:, :, :CH]
    cb_hi = codebook[:, :, CH:]
    sk2d = jnp.stack([srcp.reshape(-1, 128), kidp.reshape(-1, 128)],
                     axis=1).reshape(-1, 128)
    alo, ahi = _sc_main(vlo, vhi, sk2d, dst2d, w0, w1,
                        cb_lo, cb_hi, zeros2)
    return _tc4(alo, ahi, W_out, g_o, b_o, x)
